# baseline (speedup 1.0000x reference)
import functools

import jax
import jax.numpy as jnp
from jax import lax
from jax.experimental import pallas as pl
from jax.experimental.pallas import tpu as pltpu
from jax.experimental.pallas import tpu_sc as plsc

B, H, D, W = 8, 192, 192, 192
G, P = 36, 1024
NC, NS = 2, 16
NW = NC * NS
RAYS = B * G
RPW = RAYS // NW
L = 16
NCHUNK = P // L
CPR = P // 128
RES = 2.0

_OFFS = tuple((k & 1) + ((k >> 1) & 1) * W + ((k >> 2) & 1) * (D * W)
              for k in range(8))


def _fire_gathers(ct, idxr, gatr, c, sem):
    handles = [
        pltpu.async_copy(ct.at[idxr.at[k * CPR + c]],
                         gatr.at[k * CPR + c], sem)
        for k in range(8)
    ]
    for h in handles:
        h.wait()


def _worker_id():
    return lax.axis_index("s") * NC + lax.axis_index("c")


def _sc_body(ct, xs, ys, zs, out,
             xv, yv, zv, xdr, ydr, zdr, idxr, gatr, outv, sem):
    wid = _worker_id()

    def per_ray(r, _):
        ray = wid * RPW + r
        b = ray // G
        g = ray - b * G
        boff = b * (H * D * W)

        pltpu.sync_copy(xs.at[g], xv)
        pltpu.sync_copy(ys.at[g], yv)
        pltpu.sync_copy(zs.at[g], zv)

        inv = jnp.float32(1.0 / (P - 1))
        xh, xt = xv[pl.ds(0, L)], xv[pl.ds(P - L, L)]
        yh, yt = yv[pl.ds(0, L)], yv[pl.ds(P - L, L)]
        zh, zt = zv[pl.ds(0, L)], zv[pl.ds(P - L, L)]
        dx = (xt[L - 1] - xh[0]) * inv * RES
        dy = (yt[L - 1] - yh[0]) * inv * RES
        dz = (zt[L - 1] - zh[0]) * inv * RES
        sq = dx * dx + dy * dy + dz * dz
        z = jnp.float32(1.75)
        for _i in range(4):
            z = z * (1.5 - 0.5 * sq * z * z)
        step = sq * z
        half = step * jnp.float32(0.5)

        def idx_pass(i, carry):
            st = i * L
            x = jnp.minimum(jnp.maximum(xv[pl.ds(st, L)], 0.0), W - 1.0)
            yy = jnp.minimum(jnp.maximum(yv[pl.ds(st, L)], 0.0), D - 1.0)
            z = jnp.minimum(jnp.maximum(zv[pl.ds(st, L)], 0.0), H - 1.0)
            xi = jnp.minimum(x.astype(jnp.int32), W - 2)
            yi = jnp.minimum(yy.astype(jnp.int32), D - 2)
            zi = jnp.minimum(z.astype(jnp.int32), H - 2)
            xdr[pl.ds(st, L)] = x - xi.astype(jnp.float32)
            ydr[pl.ds(st, L)] = yy - yi.astype(jnp.float32)
            zdr[pl.ds(st, L)] = z - zi.astype(jnp.float32)
            base = zi * (D * W) + yi * W + xi + boff
            row = i // 8
            col = (i - row * 8) * L
            for k in range(8):
                idxr[k * CPR + row, pl.ds(col, L)] = base + _OFFS[k]
            return carry

        lax.fori_loop(0, NCHUNK, idx_pass, 0)

        def gather_pass(c, carry):
            _fire_gathers(ct, idxr, gatr, c, sem)
            return carry

        lax.fori_loop(0, CPR, gather_pass, 0)

        def interp_pass(i, cum):
            st = i * L
            row = i // 8
            col = (i - row * 8) * L
            cs = [gatr[k * CPR + row, pl.ds(col, L)] for k in range(8)]
            xd = xdr[pl.ds(st, L)]
            yd = ydr[pl.ds(st, L)]
            zd = zdr[pl.ds(st, L)]
            c00 = cs[0] + xd * (cs[1] - cs[0])
            c01 = cs[2] + xd * (cs[3] - cs[2])
            c10 = cs[4] + xd * (cs[5] - cs[4])
            c11 = cs[6] + xd * (cs[7] - cs[6])
            c0 = c00 + yd * (c01 - c00)
            c1 = c10 + yd * (c11 - c10)
            den = c0 + zd * (c1 - c0)
            pre = plsc.cumsum(den) + cum
            outv[pl.ds(st, L)] = pre * step + den * half
            return pre[L - 1]

        lax.fori_loop(0, NCHUNK, interp_pass, jnp.float32(0.0))

        pltpu.sync_copy(outv, out.at[ray])
        return _

    lax.fori_loop(0, RPW, per_ray, 0)


@jax.jit
def _sc_call(ct_flat, xs, ys, zs):
    mesh = plsc.VectorSubcoreMesh(core_axis_name="c", subcore_axis_name="s",
                                  num_cores=NC, num_subcores=NS)
    return pl.kernel(
        _sc_body,
        out_type=jax.ShapeDtypeStruct((RAYS, P), jnp.float32),
        mesh=mesh,
        scratch_types=[
            pltpu.VMEM((P,), jnp.float32),
            pltpu.VMEM((P,), jnp.float32),
            pltpu.VMEM((P,), jnp.float32),
            pltpu.VMEM((P,), jnp.float32),
            pltpu.VMEM((P,), jnp.float32),
            pltpu.VMEM((P,), jnp.float32),
            pltpu.VMEM((8 * CPR, 128), jnp.int32),
            pltpu.VMEM((8 * CPR, 128), jnp.float32),
            pltpu.VMEM((P,), jnp.float32),
            pltpu.SemaphoreType.DMA,
        ],
        compiler_params=pltpu.CompilerParams(needs_layout_passes=False),
    )(ct_flat, xs, ys, zs)


def kernel(ct_stack, stacked_indices):
    ct_flat = ct_stack.reshape(-1)
    coords = stacked_indices[0]
    xs = coords[:, :, 0]
    ys = coords[:, :, 1]
    zs = coords[:, :, 2]
    out = _sc_call(ct_flat, xs, ys, zs)
    return out.reshape(RAYS, P, 1)


# speedup vs baseline: 1.1358x; 1.1358x over previous
import functools

import jax
import jax.numpy as jnp
from jax import lax
from jax.experimental import pallas as pl
from jax.experimental.pallas import tpu as pltpu
from jax.experimental.pallas import tpu_sc as plsc

B, H, D, W = 8, 192, 192, 192
G, P = 36, 1024
NC, NS = 2, 16
NW = NC * NS
RAYS = B * G
RPW = RAYS // NW
L = 16
NCHUNK = P // L
CPR = P // 128
RES = 2.0

_OFFS = tuple((k & 1) + ((k >> 1) & 1) * W + ((k >> 2) & 1) * (D * W)
              for k in range(8))


def _fire_gathers(ct, idxr, gatr, grp, sems):
    for k in range(8):
        pltpu.async_copy(ct.at[idxr.at[grp * 8 + k]],
                         gatr.at[grp * 8 + k], sems.at[grp])


def _drain_gathers(ct, outv, grp, sems):
    pltpu.make_async_copy(ct.at[pl.ds(0, P)], outv, sems.at[grp]).wait()


def _worker_id():
    return lax.axis_index("s") * NC + lax.axis_index("c")


def _sc_body(ct, xs, ys, zs, out,
             xv, yv, zv, xdr, ydr, zdr, idxr, gatr, outv, sems):
    wid = _worker_id()

    def per_ray(r, _):
        ray = wid * RPW + r
        b = ray // G
        g = ray - b * G
        boff = b * (H * D * W)

        pltpu.sync_copy(xs.at[g], xv)
        pltpu.sync_copy(ys.at[g], yv)
        pltpu.sync_copy(zs.at[g], zv)

        inv = jnp.float32(1.0 / (P - 1))
        xh, xt = xv[pl.ds(0, L)], xv[pl.ds(P - L, L)]
        yh, yt = yv[pl.ds(0, L)], yv[pl.ds(P - L, L)]
        zh, zt = zv[pl.ds(0, L)], zv[pl.ds(P - L, L)]
        dx = (xt[L - 1] - xh[0]) * inv * RES
        dy = (yt[L - 1] - yh[0]) * inv * RES
        dz = (zt[L - 1] - zh[0]) * inv * RES
        sq = dx * dx + dy * dy + dz * dz
        z = jnp.float32(1.75)
        for _i in range(4):
            z = z * (1.5 - 0.5 * sq * z * z)
        step = sq * z
        half = step * jnp.float32(0.5)

        def idx_chunk(i, carry):
            st = i * L
            x = jnp.minimum(jnp.maximum(xv[pl.ds(st, L)], 0.0), W - 1.0)
            yy = jnp.minimum(jnp.maximum(yv[pl.ds(st, L)], 0.0), D - 1.0)
            z = jnp.minimum(jnp.maximum(zv[pl.ds(st, L)], 0.0), H - 1.0)
            xi = jnp.minimum(x.astype(jnp.int32), W - 2)
            yi = jnp.minimum(yy.astype(jnp.int32), D - 2)
            zi = jnp.minimum(z.astype(jnp.int32), H - 2)
            xdr[pl.ds(st, L)] = x - xi.astype(jnp.float32)
            ydr[pl.ds(st, L)] = yy - yi.astype(jnp.float32)
            zdr[pl.ds(st, L)] = z - zi.astype(jnp.float32)
            base = zi * (D * W) + yi * W + xi + boff
            grp = i // 8
            col = (i - grp * 8) * L
            for k in range(8):
                idxr[grp * 8 + k, pl.ds(col, L)] = base + _OFFS[k]
            return carry

        def idx_group(grp, carry):
            lax.fori_loop(grp * 8, grp * 8 + 8, idx_chunk, 0)
            _fire_gathers(ct, idxr, gatr, grp, sems)
            return carry

        lax.fori_loop(0, CPR, idx_group, 0)

        def interp_chunk(i, cum):
            st = i * L
            grp = i // 8
            col = (i - grp * 8) * L
            cs = [gatr[grp * 8 + k, pl.ds(col, L)] for k in range(8)]
            xd = xdr[pl.ds(st, L)]
            yd = ydr[pl.ds(st, L)]
            zd = zdr[pl.ds(st, L)]
            c00 = cs[0] + xd * (cs[1] - cs[0])
            c01 = cs[2] + xd * (cs[3] - cs[2])
            c10 = cs[4] + xd * (cs[5] - cs[4])
            c11 = cs[6] + xd * (cs[7] - cs[6])
            c0 = c00 + yd * (c01 - c00)
            c1 = c10 + yd * (c11 - c10)
            den = c0 + zd * (c1 - c0)
            pre = plsc.cumsum(den) + cum
            outv[pl.ds(st, L)] = pre * step + den * half
            return pre[L - 1]

        def interp_group(grp, cum):
            _drain_gathers(ct, outv, grp, sems)
            return lax.fori_loop(grp * 8, grp * 8 + 8, interp_chunk, cum)

        lax.fori_loop(0, CPR, interp_group, jnp.float32(0.0))

        pltpu.sync_copy(outv, out.at[ray])
        return _

    lax.fori_loop(0, RPW, per_ray, 0)


@jax.jit
def _sc_call(ct_flat, xs, ys, zs):
    mesh = plsc.VectorSubcoreMesh(core_axis_name="c", subcore_axis_name="s",
                                  num_cores=NC, num_subcores=NS)
    return pl.kernel(
        _sc_body,
        out_type=jax.ShapeDtypeStruct((RAYS, P), jnp.float32),
        mesh=mesh,
        scratch_types=[
            pltpu.VMEM((P,), jnp.float32),
            pltpu.VMEM((P,), jnp.float32),
            pltpu.VMEM((P,), jnp.float32),
            pltpu.VMEM((P,), jnp.float32),
            pltpu.VMEM((P,), jnp.float32),
            pltpu.VMEM((P,), jnp.float32),
            pltpu.VMEM((8 * CPR, 128), jnp.int32),
            pltpu.VMEM((8 * CPR, 128), jnp.float32),
            pltpu.VMEM((P,), jnp.float32),
            pltpu.SemaphoreType.DMA((CPR,)),
        ],
        compiler_params=pltpu.CompilerParams(needs_layout_passes=False),
    )(ct_flat, xs, ys, zs)


def kernel(ct_stack, stacked_indices):
    ct_flat = ct_stack.reshape(-1)
    coords = stacked_indices[0]
    xs = coords[:, :, 0]
    ys = coords[:, :, 1]
    zs = coords[:, :, 2]
    out = _sc_call(ct_flat, xs, ys, zs)
    return out.reshape(RAYS, P, 1)


# speedup vs baseline: 1.1426x; 1.0060x over previous
import functools

import jax
import jax.numpy as jnp
from jax import lax
from jax.experimental import pallas as pl
from jax.experimental.pallas import tpu as pltpu
from jax.experimental.pallas import tpu_sc as plsc

B, H, D, W = 8, 192, 192, 192
G, P = 36, 1024
NC, NS = 2, 16
NW = NC * NS
RAYS = B * G
RPW = RAYS // NW
L = 16
NCHUNK = P // L
RES = 2.0

_OFFS = tuple((k & 1) + ((k >> 1) & 1) * W + ((k >> 2) & 1) * (D * W)
              for k in range(8))


def _fire_gathers(ct, idxs, gats, sems, buf):
    for k in range(8):
        pltpu.async_copy(ct.at[idxs[k]], gats[k], sems.at[buf])


def _drain_gathers(ct, gats, sems, buf):
    for k in range(8):
        pltpu.make_async_copy(ct.at[pl.ds(0, P)], gats[k],
                              sems.at[buf]).wait()


def _worker_id():
    return lax.axis_index("s") * NC + lax.axis_index("c")


def _sc_body(ct, xs, ys, zs, out, *refs):
    xv, yv, zv, xdr, ydr, zdr = refs[0:6]
    idx_bufs = (refs[6:14], refs[14:22])
    gat_bufs = (refs[22:30], refs[30:38])
    outv = refs[38]
    sems = refs[39]
    wid = _worker_id()

    def load_and_fire(ray, buf):
        b = ray // G
        g = ray - b * G
        boff = b * (H * D * W)
        idxs = idx_bufs[buf]

        pltpu.sync_copy(xs.at[g], xv.at[buf])
        pltpu.sync_copy(ys.at[g], yv.at[buf])
        pltpu.sync_copy(zs.at[g], zv.at[buf])

        def idx_chunk(i, carry):
            st = i * L
            x = jnp.minimum(jnp.maximum(xv[buf, pl.ds(st, L)], 0.0), W - 1.0)
            yy = jnp.minimum(jnp.maximum(yv[buf, pl.ds(st, L)], 0.0), D - 1.0)
            z = jnp.minimum(jnp.maximum(zv[buf, pl.ds(st, L)], 0.0), H - 1.0)
            xi = jnp.minimum(x.astype(jnp.int32), W - 2)
            yi = jnp.minimum(yy.astype(jnp.int32), D - 2)
            zi = jnp.minimum(z.astype(jnp.int32), H - 2)
            xdr[buf, pl.ds(st, L)] = x - xi.astype(jnp.float32)
            ydr[buf, pl.ds(st, L)] = yy - yi.astype(jnp.float32)
            zdr[buf, pl.ds(st, L)] = z - zi.astype(jnp.float32)
            base = zi * (D * W) + yi * W + xi + boff
            for k in range(8):
                idxs[k][pl.ds(st, L)] = base + _OFFS[k]
            return carry

        lax.fori_loop(0, NCHUNK, idx_chunk, 0)
        _fire_gathers(ct, idxs, gat_bufs[buf], sems, buf)

    def interp_ray(ray, buf):
        gats = gat_bufs[buf]
        inv = jnp.float32(1.0 / (P - 1))
        xh, xt = xv[buf, pl.ds(0, L)], xv[buf, pl.ds(P - L, L)]
        yh, yt = yv[buf, pl.ds(0, L)], yv[buf, pl.ds(P - L, L)]
        zh, zt = zv[buf, pl.ds(0, L)], zv[buf, pl.ds(P - L, L)]
        dx = (xt[L - 1] - xh[0]) * inv * RES
        dy = (yt[L - 1] - yh[0]) * inv * RES
        dz = (zt[L - 1] - zh[0]) * inv * RES
        sq = dx * dx + dy * dy + dz * dz
        z = jnp.float32(1.75)
        for _i in range(4):
            z = z * (1.5 - 0.5 * sq * z * z)
        step = sq * z
        half = step * jnp.float32(0.5)

        def interp_chunk(i, cum):
            st = i * L
            cs = [gats[k][pl.ds(st, L)] for k in range(8)]
            xd = xdr[buf, pl.ds(st, L)]
            yd = ydr[buf, pl.ds(st, L)]
            zd = zdr[buf, pl.ds(st, L)]
            c00 = cs[0] + xd * (cs[1] - cs[0])
            c01 = cs[2] + xd * (cs[3] - cs[2])
            c10 = cs[4] + xd * (cs[5] - cs[4])
            c11 = cs[6] + xd * (cs[7] - cs[6])
            c0 = c00 + yd * (c01 - c00)
            c1 = c10 + yd * (c11 - c10)
            den = c0 + zd * (c1 - c0)
            pre = plsc.cumsum(den) + cum
            outv[pl.ds(st, L)] = pre * step + den * half
            return pre[L - 1]

        lax.fori_loop(0, NCHUNK, interp_chunk, jnp.float32(0.0))
        pltpu.sync_copy(outv, out.at[ray])

    ray0 = wid * RPW
    load_and_fire(ray0, 0)
    for r in range(RPW):
        buf = r & 1
        if r + 1 < RPW:
            load_and_fire(ray0 + (r + 1), 1 - buf)
        _drain_gathers(ct, gat_bufs[buf], sems, buf)
        interp_ray(ray0 + r, buf)


@jax.jit
def _sc_call(ct_flat, xs, ys, zs):
    mesh = plsc.VectorSubcoreMesh(core_axis_name="c", subcore_axis_name="s",
                                  num_cores=NC, num_subcores=NS)
    return pl.kernel(
        _sc_body,
        out_type=jax.ShapeDtypeStruct((RAYS, P), jnp.float32),
        mesh=mesh,
        scratch_types=(
            [pltpu.VMEM((2, P), jnp.float32)] * 6
            + [pltpu.VMEM((P,), jnp.int32)] * 16
            + [pltpu.VMEM((P,), jnp.float32)] * 16
            + [pltpu.VMEM((P,), jnp.float32)]
            + [pltpu.SemaphoreType.DMA((2,))]
        ),
        compiler_params=pltpu.CompilerParams(needs_layout_passes=False),
    )(ct_flat, xs, ys, zs)


def kernel(ct_stack, stacked_indices):
    ct_flat = ct_stack.reshape(-1)
    coords = stacked_indices[0]
    xs = coords[:, :, 0]
    ys = coords[:, :, 1]
    zs = coords[:, :, 2]
    out = _sc_call(ct_flat, xs, ys, zs)
    return out.reshape(RAYS, P, 1)
